# Initial kernel scaffold; baseline (speedup 1.0000x reference)
import jax
import jax.numpy as jnp
from jax.experimental import pallas as pl


def kernel(x, edge_index, W_ih, W_hh, b_r, W_fc, b_fc):
    raise NotImplementedError("write your pallas kernel here")


# speedup vs baseline: 9.0616x; 9.0616x over previous
import functools

import jax
import jax.numpy as jnp
from jax import lax
from jax.experimental import pallas as pl
from jax.experimental.pallas import tpu as pltpu
from jax.experimental.pallas import tpu_sc as plsc

N, E, D, H, O = 10000, 320000, 128, 64, 128
NC, NS = 2, 16
NW = NC * NS
EPW = E // NW
CK = 80
NCHUNK = EPW // CK
NP = 10240
RPT = NP // NS
RB = 128
NRB = RPT // RB
RT = 1000


def _pre_body(x_ref, wih_ref, whh_ref, br_ref, p_ref, xi_ref):
    xb = x_ref[...]
    p_ref[...] = jnp.dot(xb, whh_ref[...], preferred_element_type=jnp.float32)
    xi_ref[...] = (
        jnp.dot(xb, wih_ref[...], preferred_element_type=jnp.float32) + br_ref[...]
    )


def _post_body(xi_ref, acc_ref, wfc_ref, bfc_ref, out_ref):
    h = jnp.tanh(xi_ref[...] + acc_ref[0] + acc_ref[1])
    out_ref[...] = (
        jnp.dot(h, wfc_ref[...], preferred_element_type=jnp.float32) + bfc_ref[...]
    )


def _sc_body(p_hbm, src_hbm, dst_hbm, out_hbm, src_v, dst_v, rows_v, zbuf_v, acc_sh, sem):
    c = lax.axis_index("c")
    s = lax.axis_index("s")
    wid = c * NS + s

    def _zrow(r, carry):
        for q in range(H // 16):
            zbuf_v[r, pl.ds(q * 16, 16)] = jnp.zeros((16,), jnp.float32)
        return carry

    lax.fori_loop(0, RB, _zrow, 0)
    row0 = s * RPT
    for b in range(NRB):
        pltpu.sync_copy(zbuf_v, acc_sh.at[pl.ds(row0 + b * RB, RB)])

    pltpu.sync_copy(src_hbm.at[wid], src_v)
    pltpu.sync_copy(dst_hbm.at[wid], dst_v)
    plsc.subcore_barrier()

    def _chunk(j, carry):
        pltpu.async_copy(p_hbm.at[src_v.at[j]], rows_v, sem).wait()
        pltpu.sync_copy(rows_v, acc_sh.at[dst_v.at[j]], add=True)
        return carry

    lax.fori_loop(0, NCHUNK, _chunk, 0)

    plsc.subcore_barrier()
    for b in range(NRB):
        r = row0 + b * RB
        pltpu.sync_copy(acc_sh.at[pl.ds(r, RB)], zbuf_v)
        pltpu.sync_copy(zbuf_v, out_hbm.at[c, pl.ds(r, RB)])


_sc_accum = functools.partial(
    pl.kernel,
    out_type=jax.ShapeDtypeStruct((NC, NP, H), jnp.float32),
    mesh=plsc.VectorSubcoreMesh(core_axis_name="c", subcore_axis_name="s"),
    scratch_types=[
        pltpu.VMEM((NCHUNK, CK), jnp.int32),
        pltpu.VMEM((NCHUNK, CK), jnp.int32),
        pltpu.VMEM((CK, H), jnp.float32),
        pltpu.VMEM((RB, H), jnp.float32),
        pltpu.VMEM_SHARED((NP, H), jnp.float32),
        pltpu.SemaphoreType.DMA,
    ],
    compiler_params=pltpu.CompilerParams(use_tc_tiling_on_sc=False),
)(_sc_body)


def kernel(x, edge_index, W_ih, W_hh, b_r, W_fc, b_fc):
    src = edge_index[0].astype(jnp.int32).reshape(NW, NCHUNK, CK)
    dst = edge_index[1].astype(jnp.int32).reshape(NW, NCHUNK, CK)

    p, xi = pl.pallas_call(
        _pre_body,
        grid=(N // RT,),
        in_specs=[
            pl.BlockSpec((RT, D), lambda i: (i, 0)),
            pl.BlockSpec((D, H), lambda i: (0, 0)),
            pl.BlockSpec((D, H), lambda i: (0, 0)),
            pl.BlockSpec((1, H), lambda i: (0, 0)),
        ],
        out_specs=[
            pl.BlockSpec((RT, H), lambda i: (i, 0)),
            pl.BlockSpec((RT, H), lambda i: (i, 0)),
        ],
        out_shape=[
            jax.ShapeDtypeStruct((N, H), jnp.float32),
            jax.ShapeDtypeStruct((N, H), jnp.float32),
        ],
    )(x, W_ih, W_hh, b_r.reshape(1, H))

    partials = _sc_accum(p, src, dst)

    out = pl.pallas_call(
        _post_body,
        grid=(N // RT,),
        in_specs=[
            pl.BlockSpec((RT, H), lambda i: (i, 0)),
            pl.BlockSpec((NC, RT, H), lambda i: (0, i, 0)),
            pl.BlockSpec((H, O), lambda i: (0, 0)),
            pl.BlockSpec((1, O), lambda i: (0, 0)),
        ],
        out_specs=pl.BlockSpec((RT, O), lambda i: (i, 0)),
        out_shape=jax.ShapeDtypeStruct((N, O), jnp.float32),
    )(xi, partials, W_fc, b_fc.reshape(1, O))
    return out


# speedup vs baseline: 12.9033x; 1.4240x over previous
import functools

import jax
import jax.numpy as jnp
from jax import lax
from jax.experimental import pallas as pl
from jax.experimental.pallas import tpu as pltpu
from jax.experimental.pallas import tpu_sc as plsc

N, E, D, H, O = 10000, 320000, 128, 64, 128
NC, NS = 2, 16
NW = NC * NS
EPW = E // NW
CK = 80
NCHUNK = EPW // CK
NP = 10240
RPT = NP // NS
RB = 128
NRB = RPT // RB
RT = 1000


def _pre_body(x_ref, wih_ref, whh_ref, br_ref, p_ref, xi_ref):
    xb = x_ref[...]
    p_ref[...] = jnp.dot(xb, whh_ref[...], preferred_element_type=jnp.float32)
    xi_ref[...] = (
        jnp.dot(xb, wih_ref[...], preferred_element_type=jnp.float32) + br_ref[...]
    )


def _post_body(xi_ref, acc_ref, wfc_ref, bfc_ref, out_ref):
    h = jnp.tanh(xi_ref[...] + acc_ref[0] + acc_ref[1])
    out_ref[...] = (
        jnp.dot(h, wfc_ref[...], preferred_element_type=jnp.float32) + bfc_ref[...]
    )


def _sc_body(p_hbm, src_hbm, dst_hbm, out_hbm, src_v, dst_v, rows0_v, rows1_v,
             zbuf_v, acc_sh, sem0, sem1):
    c = lax.axis_index("c")
    s = lax.axis_index("s")
    wid = c * NS + s

    def _zrow(r, carry):
        for q in range(H // 16):
            zbuf_v[r, pl.ds(q * 16, 16)] = jnp.zeros((16,), jnp.float32)
        return carry

    lax.fori_loop(0, RB, _zrow, 0)
    row0 = s * RPT
    for b in range(NRB):
        pltpu.sync_copy(zbuf_v, acc_sh.at[pl.ds(row0 + b * RB, RB)])

    pltpu.sync_copy(src_hbm.at[wid], src_v)
    pltpu.sync_copy(dst_hbm.at[wid], dst_v)
    plsc.subcore_barrier()

    pltpu.async_copy(p_hbm.at[src_v.at[0]], rows0_v, sem0)

    def _pair(i, carry):
        j0 = 2 * i
        pltpu.async_copy(p_hbm.at[src_v.at[j0 + 1]], rows1_v, sem1)
        pltpu.make_async_copy(p_hbm.at[src_v.at[j0]], rows0_v, sem0).wait()
        pltpu.sync_copy(rows0_v, acc_sh.at[dst_v.at[j0]], add=True)

        @pl.when(j0 + 2 < NCHUNK)
        def _():
            pltpu.async_copy(p_hbm.at[src_v.at[j0 + 2]], rows0_v, sem0)

        pltpu.make_async_copy(p_hbm.at[src_v.at[j0 + 1]], rows1_v, sem1).wait()
        pltpu.sync_copy(rows1_v, acc_sh.at[dst_v.at[j0 + 1]], add=True)
        return carry

    lax.fori_loop(0, NCHUNK // 2, _pair, 0)
    if NCHUNK % 2:
        jt = NCHUNK - 1
        pltpu.make_async_copy(p_hbm.at[src_v.at[jt]], rows0_v, sem0).wait()
        pltpu.sync_copy(rows0_v, acc_sh.at[dst_v.at[jt]], add=True)

    plsc.subcore_barrier()
    for b in range(NRB):
        r = row0 + b * RB
        pltpu.sync_copy(acc_sh.at[pl.ds(r, RB)], zbuf_v)
        pltpu.sync_copy(zbuf_v, out_hbm.at[c, pl.ds(r, RB)])


_sc_accum = functools.partial(
    pl.kernel,
    out_type=jax.ShapeDtypeStruct((NC, NP, H), jnp.float32),
    mesh=plsc.VectorSubcoreMesh(core_axis_name="c", subcore_axis_name="s"),
    scratch_types=[
        pltpu.VMEM((NCHUNK, CK), jnp.int32),
        pltpu.VMEM((NCHUNK, CK), jnp.int32),
        pltpu.VMEM((CK, H), jnp.float32),
        pltpu.VMEM((CK, H), jnp.float32),
        pltpu.VMEM((RB, H), jnp.float32),
        pltpu.VMEM_SHARED((NP, H), jnp.float32),
        pltpu.SemaphoreType.DMA,
        pltpu.SemaphoreType.DMA,
    ],
    compiler_params=pltpu.CompilerParams(use_tc_tiling_on_sc=False),
)(_sc_body)


def kernel(x, edge_index, W_ih, W_hh, b_r, W_fc, b_fc):
    src = edge_index[0].astype(jnp.int32).reshape(NW, NCHUNK, CK)
    dst = edge_index[1].astype(jnp.int32).reshape(NW, NCHUNK, CK)

    p, xi = pl.pallas_call(
        _pre_body,
        grid=(N // RT,),
        in_specs=[
            pl.BlockSpec((RT, D), lambda i: (i, 0)),
            pl.BlockSpec((D, H), lambda i: (0, 0)),
            pl.BlockSpec((D, H), lambda i: (0, 0)),
            pl.BlockSpec((1, H), lambda i: (0, 0)),
        ],
        out_specs=[
            pl.BlockSpec((RT, H), lambda i: (i, 0)),
            pl.BlockSpec((RT, H), lambda i: (i, 0)),
        ],
        out_shape=[
            jax.ShapeDtypeStruct((N, H), jnp.float32),
            jax.ShapeDtypeStruct((N, H), jnp.float32),
        ],
    )(x, W_ih, W_hh, b_r.reshape(1, H))

    partials = _sc_accum(p, src, dst)

    out = pl.pallas_call(
        _post_body,
        grid=(N // RT,),
        in_specs=[
            pl.BlockSpec((RT, H), lambda i: (i, 0)),
            pl.BlockSpec((NC, RT, H), lambda i: (0, i, 0)),
            pl.BlockSpec((H, O), lambda i: (0, 0)),
            pl.BlockSpec((1, O), lambda i: (0, 0)),
        ],
        out_specs=pl.BlockSpec((RT, O), lambda i: (i, 0)),
        out_shape=jax.ShapeDtypeStruct((N, O), jnp.float32),
    )(xi, partials, W_fc, b_fc.reshape(1, O))
    return out
